# Initial kernel scaffold; baseline (speedup 1.0000x reference)
import jax
import jax.numpy as jnp
from jax.experimental import pallas as pl


def kernel(y, hist_average_y, avg_speed, time_feats, edge_attr, emb, W1, b1, W2, b2, Wd1, bd1, Wd2, bd2, edge_index):
    raise NotImplementedError("write your pallas kernel here")


# speedup vs baseline: 50.0579x; 50.0579x over previous
import functools

import jax
import jax.numpy as jnp
from jax import lax
from jax.experimental import pallas as pl
from jax.experimental.pallas import tpu as pltpu
from jax.experimental.pallas import tpu_sc as plsc

SEQ = 4
PACK = 48
GW = 128
SW = 2000
MLP_B = 2560


def _gather_call(packed, emb, src2d, dst2d):
    n_edges = src2d.shape[1]
    mesh = plsc.VectorSubcoreMesh(core_axis_name="core", subcore_axis_name="subcore")

    @functools.partial(
        pl.kernel,
        out_type=(
            jax.ShapeDtypeStruct((n_edges, PACK), jnp.float32),
            jax.ShapeDtypeStruct((n_edges, 32), jnp.float32),
        ),
        mesh=mesh,
        compiler_params=pltpu.CompilerParams(use_tc_tiling_on_sc=False),
    )
    def gather_kernel(packed_hbm, emb_hbm, src_hbm, dst_hbm, o1_hbm, o2_hbm):
        def body(si, di, o1, o2):
            pltpu.sync_copy(packed_hbm.at[si.at[0]], o1)
            pltpu.sync_copy(emb_hbm.at[di.at[0]], o2)

        pltpu.emit_pipeline(
            body,
            grid=(n_edges // GW,),
            in_specs=[
                pl.BlockSpec((1, GW), lambda i: (0, i)),
                pl.BlockSpec((1, GW), lambda i: (0, i)),
            ],
            out_specs=[
                pl.BlockSpec((GW, PACK), lambda i: (i, 0)),
                pl.BlockSpec((GW, 32), lambda i: (i, 0)),
            ],
            core_axis_name=("core", "subcore"),
            dimension_semantics=(pltpu.PARALLEL,),
        )(src_hbm, dst_hbm, o1_hbm, o2_hbm)

    return gather_kernel(packed, emb, src2d, dst2d)


def _mlp_body(sp_ref, ed_ref, ea_ref, w1p_ref, wav_ref, wea_ref, r_ref, w2_ref,
              p_ref, q_ref):
    sp = sp_ref[...]
    prod = sp[:, 0:32] * ed_ref[...]
    base = jnp.dot(prod, w1p_ref[...], preferred_element_type=jnp.float32,
                   precision=lax.Precision.HIGHEST)
    base = base + ea_ref[...] * wea_ref[...][None, :]
    w2 = w2_ref[...][None, :]
    wav = wav_ref[...][None, :]
    p_acc = jnp.zeros_like(ea_ref[...])
    q_acc = jnp.zeros_like(p_acc)
    for s in range(SEQ):
        av_s = sp[:, 32 + s:33 + s]
        x_s = sp[:, 36 + s:37 + s]
        h = jnp.maximum(base + av_s * wav + r_ref[s:s + 1, :], 0.0)
        logit = jnp.sum(h * w2, axis=1, keepdims=True)
        ex = jnp.exp(logit)
        p_acc = p_acc + ex
        q_acc = q_acc + ex * x_s
    p_ref[...] = p_acc
    q_ref[...] = q_acc


def _mlp_call(src_rows, emb_dst, edge_attr, w1p, wav, wea, r, w2):
    n_edges = src_rows.shape[0]
    grid = (n_edges // MLP_B,)
    full = lambda shape: pl.BlockSpec(shape, lambda i: tuple(0 for _ in shape))
    return pl.pallas_call(
        _mlp_body,
        grid=grid,
        in_specs=[
            pl.BlockSpec((MLP_B, PACK), lambda i: (i, 0)),
            pl.BlockSpec((MLP_B, 32), lambda i: (i, 0)),
            pl.BlockSpec((MLP_B, 1), lambda i: (i, 0)),
            full((32, 64)),
            full((64,)),
            full((64,)),
            full((SEQ, 64)),
            full((64,)),
        ],
        out_specs=[
            pl.BlockSpec((MLP_B, 1), lambda i: (i, 0)),
            pl.BlockSpec((MLP_B, 1), lambda i: (i, 0)),
        ],
        out_shape=(
            jax.ShapeDtypeStruct((n_edges, 1), jnp.float32),
            jax.ShapeDtypeStruct((n_edges, 1), jnp.float32),
        ),
    )(src_rows, emb_dst, edge_attr, w1p, wav, wea, r, w2)


def _scatter_call(dst_g, p_g, q_g, n_nodes):
    n_win, _ = dst_g.shape
    n_workers = 32
    acc_len = 2 * n_nodes
    mesh = plsc.VectorSubcoreMesh(core_axis_name="core", subcore_axis_name="subcore")

    @functools.partial(
        pl.kernel,
        out_type=jax.ShapeDtypeStruct((n_workers, acc_len), jnp.float32),
        mesh=mesh,
        scratch_types=[pltpu.VMEM((acc_len,), jnp.float32)],
        compiler_params=pltpu.CompilerParams(needs_layout_passes=False),
    )
    def scatter_kernel(dst_hbm, p_hbm, q_hbm, out_hbm, acc):
        wid = lax.axis_index("subcore") * 2 + lax.axis_index("core")

        @pl.loop(0, acc_len, step=16)
        def _(i):
            acc[pl.ds(i, 16)] = jnp.zeros((16,), jnp.float32)

        def body(dv, pv, qv):
            @pl.loop(0, SW, step=16)
            def _(j):
                idx = dv[0, pl.ds(j, 16)]
                plsc.addupdate_scatter(acc, [idx], pv[0, pl.ds(j, 16)])
                plsc.addupdate_scatter(acc, [idx + n_nodes], qv[0, pl.ds(j, 16)])

        pltpu.emit_pipeline(
            body,
            grid=(n_win,),
            in_specs=[
                pl.BlockSpec((1, SW), lambda i: (i, 0)),
                pl.BlockSpec((1, SW), lambda i: (i, 0)),
                pl.BlockSpec((1, SW), lambda i: (i, 0)),
            ],
            core_axis_name=("core", "subcore"),
            dimension_semantics=(pltpu.PARALLEL,),
        )(dst_hbm, p_hbm, q_hbm)

        pltpu.sync_copy(acc, out_hbm.at[wid])

    return scatter_kernel(dst_g, p_g, q_g)


def _combine_body(parts_ref, out_ref):
    s = jnp.sum(parts_ref[...], axis=0)
    out_ref[...] = s[1] / (s[0] + 1e-16)


def _combine_call(parts3, n_nodes):
    n_workers = parts3.shape[0]
    return pl.pallas_call(
        _combine_body,
        in_specs=[pl.BlockSpec((n_workers, 2, n_nodes), lambda: (0, 0, 0))],
        out_specs=pl.BlockSpec((n_nodes,), lambda: (0,)),
        out_shape=jax.ShapeDtypeStruct((n_nodes,), jnp.float32),
    )(parts3)


def kernel(y, hist_average_y, avg_speed, time_feats, edge_attr, emb,
           W1, b1, W2, b2, Wd1, bd1, Wd2, bd2, edge_index):
    n_nodes = emb.shape[0]
    n_edges = edge_index.shape[1]

    x_t = y[:SEQ].T
    av_t = avg_speed[:SEQ].T
    packed = jnp.concatenate(
        [emb, av_t, x_t, jnp.zeros((n_nodes, PACK - 40), jnp.float32)], axis=1)

    src2d = edge_index[0].reshape(1, n_edges)
    dst2d = edge_index[1].reshape(1, n_edges)

    w1p = W1[6:38]
    wav = W1[0]
    wea = W1[5]
    r = W1[1:5] + b1[None, :]

    src_rows, emb_dst = _gather_call(packed, emb, src2d, dst2d)
    p, q = _mlp_call(src_rows, emb_dst, edge_attr, w1p, wav, wea, r, W2[:, 0])

    dst_g = edge_index[1].reshape(n_edges // SW, SW)
    p_g = p.reshape(n_edges // SW, SW)
    q_g = q.reshape(n_edges // SW, SW)
    parts = _scatter_call(dst_g, p_g, q_g, n_nodes)

    return _combine_call(parts.reshape(32, 2, n_nodes), n_nodes)


# speedup vs baseline: 66.4861x; 1.3282x over previous
import functools

import jax
import jax.numpy as jnp
from jax import lax
from jax.experimental import pallas as pl
from jax.experimental.pallas import tpu as pltpu
from jax.experimental.pallas import tpu_sc as plsc

SEQ = 4
PACK = 40
GW = 128
SW = 2000
MLP_B = 2560


def _gather_call(packed, emb, src2d, dst2d):
    n_edges = src2d.shape[1]
    mesh = plsc.VectorSubcoreMesh(core_axis_name="core", subcore_axis_name="subcore")

    @functools.partial(
        pl.kernel,
        out_type=(
            jax.ShapeDtypeStruct((n_edges, PACK), jnp.float32),
            jax.ShapeDtypeStruct((n_edges, 32), jnp.float32),
        ),
        mesh=mesh,
        compiler_params=pltpu.CompilerParams(use_tc_tiling_on_sc=False),
    )
    def gather_kernel(packed_hbm, emb_hbm, src_hbm, dst_hbm, o1_hbm, o2_hbm):
        def body(si, di, o1, o2):
            pltpu.sync_copy(packed_hbm.at[si.at[0]], o1)
            pltpu.sync_copy(emb_hbm.at[di.at[0]], o2)

        pltpu.emit_pipeline(
            body,
            grid=(n_edges // GW,),
            in_specs=[
                pl.BlockSpec((1, GW), lambda i: (0, i)),
                pl.BlockSpec((1, GW), lambda i: (0, i)),
            ],
            out_specs=[
                pl.BlockSpec((GW, PACK), lambda i: (i, 0)),
                pl.BlockSpec((GW, 32), lambda i: (i, 0)),
            ],
            core_axis_name=("core", "subcore"),
            dimension_semantics=(pltpu.PARALLEL,),
        )(src_hbm, dst_hbm, o1_hbm, o2_hbm)

    return gather_kernel(packed, emb, src2d, dst2d)


def _mlp_body(sp_ref, ed_ref, ea_ref, x4_ref, w4t_ref, r4_ref, w2b_ref,
              p_ref, q_ref):
    sp = sp_ref[...]
    prod = sp[:, 0:32] * ed_ref[...]
    feats = jnp.concatenate([prod, ea_ref[...], sp[:, 32:36]], axis=1)
    h4 = jnp.maximum(
        lax.dot_general(w4t_ref[...], feats, (((1,), (1,)), ((), ())),
                        preferred_element_type=jnp.float32) + r4_ref[...],
        0.0)
    logits = lax.dot_general(w2b_ref[...], h4, (((1,), (0,)), ((), ())),
                             preferred_element_type=jnp.float32)
    ex = jnp.exp(logits)
    p_ref[...] = jnp.sum(ex, axis=0, keepdims=True)
    q_ref[...] = jnp.sum(ex * x4_ref[...], axis=0, keepdims=True)


def _mlp_call(src_rows, emb_dst, ea_col, x4t, w4t, r4, w2b):
    n_edges = src_rows.shape[0]
    grid = (n_edges // MLP_B,)
    full = lambda shape: pl.BlockSpec(shape, lambda i: tuple(0 for _ in shape))
    return pl.pallas_call(
        _mlp_body,
        grid=grid,
        in_specs=[
            pl.BlockSpec((MLP_B, PACK), lambda i: (i, 0)),
            pl.BlockSpec((MLP_B, 32), lambda i: (i, 0)),
            pl.BlockSpec((MLP_B, 1), lambda i: (i, 0)),
            pl.BlockSpec((SEQ, MLP_B), lambda i: (0, i)),
            full((4 * 64, 37)),
            full((4 * 64, 1)),
            full((SEQ, 4 * 64)),
        ],
        out_specs=[
            pl.BlockSpec((1, MLP_B), lambda i: (0, i)),
            pl.BlockSpec((1, MLP_B), lambda i: (0, i)),
        ],
        out_shape=(
            jax.ShapeDtypeStruct((1, n_edges), jnp.float32),
            jax.ShapeDtypeStruct((1, n_edges), jnp.float32),
        ),
    )(src_rows, emb_dst, ea_col, x4t, w4t, r4, w2b)


def _scatter_call(dst_g, p_g, q_g, n_nodes):
    n_win, _ = dst_g.shape
    n_workers = 32
    acc_len = 2 * n_nodes
    mesh = plsc.VectorSubcoreMesh(core_axis_name="core", subcore_axis_name="subcore")

    @functools.partial(
        pl.kernel,
        out_type=jax.ShapeDtypeStruct((n_workers, acc_len), jnp.float32),
        mesh=mesh,
        scratch_types=[pltpu.VMEM((acc_len,), jnp.float32)],
        compiler_params=pltpu.CompilerParams(needs_layout_passes=False),
    )
    def scatter_kernel(dst_hbm, p_hbm, q_hbm, out_hbm, acc):
        wid = lax.axis_index("subcore") * 2 + lax.axis_index("core")

        @pl.loop(0, acc_len, step=16)
        def _(i):
            acc[pl.ds(i, 16)] = jnp.zeros((16,), jnp.float32)

        def body(dv, pv, qv):
            @pl.loop(0, SW, step=16)
            def _(j):
                idx = dv[0, pl.ds(j, 16)]
                plsc.addupdate_scatter(acc, [idx], pv[0, pl.ds(j, 16)])
                plsc.addupdate_scatter(acc, [idx + n_nodes], qv[0, pl.ds(j, 16)])

        pltpu.emit_pipeline(
            body,
            grid=(n_win,),
            in_specs=[
                pl.BlockSpec((1, SW), lambda i: (i, 0)),
                pl.BlockSpec((1, SW), lambda i: (i, 0)),
                pl.BlockSpec((1, SW), lambda i: (i, 0)),
            ],
            core_axis_name=("core", "subcore"),
            dimension_semantics=(pltpu.PARALLEL,),
        )(dst_hbm, p_hbm, q_hbm)

        pltpu.sync_copy(acc, out_hbm.at[wid])

    return scatter_kernel(dst_g, p_g, q_g)


def _combine_body(parts_ref, out_ref):
    s = jnp.sum(parts_ref[...], axis=0)
    out_ref[...] = s[1] / (s[0] + 1e-16)


def _combine_call(parts3, n_nodes):
    n_workers = parts3.shape[0]
    return pl.pallas_call(
        _combine_body,
        in_specs=[pl.BlockSpec((n_workers, 2, n_nodes), lambda: (0, 0, 0))],
        out_specs=pl.BlockSpec((n_nodes,), lambda: (0,)),
        out_shape=jax.ShapeDtypeStruct((n_nodes,), jnp.float32),
    )(parts3)


def kernel(y, hist_average_y, avg_speed, time_feats, edge_attr, emb,
           W1, b1, W2, b2, Wd1, bd1, Wd2, bd2, edge_index):
    n_nodes = emb.shape[0]
    n_edges = edge_index.shape[1]

    x_t = y[:SEQ].T
    av_t = avg_speed[:SEQ].T
    packed = jnp.concatenate([emb, av_t, x_t], axis=1)

    src2d = edge_index[0].reshape(1, n_edges)
    dst2d = edge_index[1].reshape(1, n_edges)

    w4t = jnp.zeros((SEQ * 64, 37), jnp.float32)
    r4 = jnp.zeros((SEQ * 64, 1), jnp.float32)
    w2b = jnp.zeros((SEQ, SEQ * 64), jnp.float32)
    for s in range(SEQ):
        blk = slice(s * 64, (s + 1) * 64)
        w4t = w4t.at[blk, 0:32].set(W1[6:38].T)
        w4t = w4t.at[blk, 32].set(W1[5])
        w4t = w4t.at[blk, 33 + s].set(W1[0])
        r4 = r4.at[blk, 0].set(W1[1 + s] + b1)
        w2b = w2b.at[s, blk].set(W2[:, 0])

    src_rows, emb_dst = _gather_call(packed, emb, src2d, dst2d)
    x4t = src_rows[:, 36:40].T
    p, q = _mlp_call(src_rows, emb_dst, edge_attr, x4t, w4t, r4, w2b)

    dst_g = edge_index[1].reshape(n_edges // SW, SW)
    p_g = p.reshape(n_edges // SW, SW)
    q_g = q.reshape(n_edges // SW, SW)
    parts = _scatter_call(dst_g, p_g, q_g, n_nodes)

    return _combine_call(parts.reshape(32, 2, n_nodes), n_nodes)


# speedup vs baseline: 80.7295x; 1.2142x over previous
import functools

import jax
import jax.numpy as jnp
from jax import lax
from jax.experimental import pallas as pl
from jax.experimental.pallas import tpu as pltpu
from jax.experimental.pallas import tpu_sc as plsc

SEQ = 4
PACK = 40
GW = 128
SW = 2000
MLP_B = 2560


def _gather_call(packed, emb, src2d, dst2d):
    n_edges = src2d.shape[1]
    mesh = plsc.VectorSubcoreMesh(core_axis_name="core", subcore_axis_name="subcore")

    @functools.partial(
        pl.kernel,
        out_type=(
            jax.ShapeDtypeStruct((n_edges, PACK), jnp.float32),
            jax.ShapeDtypeStruct((n_edges, 32), jnp.float32),
        ),
        mesh=mesh,
        compiler_params=pltpu.CompilerParams(use_tc_tiling_on_sc=False),
    )
    def gather_kernel(packed_hbm, emb_hbm, src_hbm, dst_hbm, o1_hbm, o2_hbm):
        def body(si, di, o1, o2):
            pltpu.sync_copy(packed_hbm.at[si.at[0]], o1)
            pltpu.sync_copy(emb_hbm.at[di.at[0]], o2)

        pltpu.emit_pipeline(
            body,
            grid=(n_edges // GW,),
            in_specs=[
                pl.BlockSpec((1, GW), lambda i: (0, i)),
                pl.BlockSpec((1, GW), lambda i: (0, i)),
            ],
            out_specs=[
                pl.BlockSpec((GW, PACK), lambda i: (i, 0)),
                pl.BlockSpec((GW, 32), lambda i: (i, 0)),
            ],
            core_axis_name=("core", "subcore"),
            dimension_semantics=(pltpu.PARALLEL,),
        )(src_hbm, dst_hbm, o1_hbm, o2_hbm)

    return gather_kernel(packed, emb, src2d, dst2d)


def _mlp_body(sp_ref, ed_ref, ea_ref, w4t_ref, r4_ref, w2b_ref,
              p_ref, q_ref):
    sp = sp_ref[...]
    prod = sp[:, 0:32] * ed_ref[...]
    feats = jnp.concatenate([prod, ea_ref[...], sp[:, 32:36]], axis=1)
    h4 = jnp.maximum(
        lax.dot_general(w4t_ref[...], feats, (((1,), (1,)), ((), ())),
                        preferred_element_type=jnp.float32) + r4_ref[...],
        0.0)
    logits = lax.dot_general(w2b_ref[...], h4, (((1,), (0,)), ((), ())),
                             preferred_element_type=jnp.float32)
    ex = jnp.exp(logits)
    x4 = jnp.transpose(sp[:, 36:40])
    p_ref[...] = jnp.sum(ex, axis=0, keepdims=True)
    q_ref[...] = jnp.sum(ex * x4, axis=0, keepdims=True)


def _mlp_call(src_rows, emb_dst, ea_col, w4t, r4, w2b):
    n_edges = src_rows.shape[0]
    grid = (n_edges // MLP_B,)
    full = lambda shape: pl.BlockSpec(shape, lambda i: tuple(0 for _ in shape))
    return pl.pallas_call(
        _mlp_body,
        grid=grid,
        in_specs=[
            pl.BlockSpec((MLP_B, PACK), lambda i: (i, 0)),
            pl.BlockSpec((MLP_B, 32), lambda i: (i, 0)),
            pl.BlockSpec((MLP_B, 1), lambda i: (i, 0)),
            full((4 * 64, 37)),
            full((4 * 64, 1)),
            full((SEQ, 4 * 64)),
        ],
        out_specs=[
            pl.BlockSpec((1, MLP_B), lambda i: (0, i)),
            pl.BlockSpec((1, MLP_B), lambda i: (0, i)),
        ],
        out_shape=(
            jax.ShapeDtypeStruct((1, n_edges), jnp.float32),
            jax.ShapeDtypeStruct((1, n_edges), jnp.float32),
        ),
    )(src_rows, emb_dst, ea_col, w4t, r4, w2b)


def _scatter_call(dst_g, p_g, q_g, n_nodes):
    n_win, _ = dst_g.shape
    n_workers = 32
    acc_len = 2 * n_nodes
    mesh = plsc.VectorSubcoreMesh(core_axis_name="core", subcore_axis_name="subcore")

    @functools.partial(
        pl.kernel,
        out_type=jax.ShapeDtypeStruct((n_workers, acc_len), jnp.float32),
        mesh=mesh,
        scratch_types=[pltpu.VMEM((acc_len,), jnp.float32)],
        compiler_params=pltpu.CompilerParams(needs_layout_passes=False),
    )
    def scatter_kernel(dst_hbm, p_hbm, q_hbm, out_hbm, acc):
        wid = lax.axis_index("subcore") * 2 + lax.axis_index("core")

        @pl.loop(0, acc_len, step=16)
        def _(i):
            acc[pl.ds(i, 16)] = jnp.zeros((16,), jnp.float32)

        def body(dv, pv, qv):
            @pl.loop(0, SW, step=16)
            def _(j):
                idx = dv[0, pl.ds(j, 16)]
                plsc.addupdate_scatter(acc, [idx], pv[0, pl.ds(j, 16)])
                plsc.addupdate_scatter(acc, [idx + n_nodes], qv[0, pl.ds(j, 16)])

        pltpu.emit_pipeline(
            body,
            grid=(n_win,),
            in_specs=[
                pl.BlockSpec((1, SW), lambda i: (i, 0)),
                pl.BlockSpec((1, SW), lambda i: (i, 0)),
                pl.BlockSpec((1, SW), lambda i: (i, 0)),
            ],
            core_axis_name=("core", "subcore"),
            dimension_semantics=(pltpu.PARALLEL,),
        )(dst_hbm, p_hbm, q_hbm)

        pltpu.sync_copy(acc, out_hbm.at[wid])

    return scatter_kernel(dst_g, p_g, q_g)


def _combine_body(parts_ref, out_ref):
    s = jnp.sum(parts_ref[...], axis=0)
    out_ref[...] = s[1] / (s[0] + 1e-16)


def _combine_call(parts3, n_nodes):
    n_workers = parts3.shape[0]
    return pl.pallas_call(
        _combine_body,
        in_specs=[pl.BlockSpec((n_workers, 2, n_nodes), lambda: (0, 0, 0))],
        out_specs=pl.BlockSpec((n_nodes,), lambda: (0,)),
        out_shape=jax.ShapeDtypeStruct((n_nodes,), jnp.float32),
    )(parts3)


def kernel(y, hist_average_y, avg_speed, time_feats, edge_attr, emb,
           W1, b1, W2, b2, Wd1, bd1, Wd2, bd2, edge_index):
    n_nodes = emb.shape[0]
    n_edges = edge_index.shape[1]

    x_t = y[:SEQ].T
    av_t = avg_speed[:SEQ].T
    packed = jnp.concatenate([emb, av_t, x_t], axis=1)

    src2d = edge_index[0].reshape(1, n_edges)
    dst2d = edge_index[1].reshape(1, n_edges)

    w4t = jnp.concatenate(
        [jnp.tile(W1[6:38].T, (SEQ, 1)),
         jnp.tile(W1[5].reshape(64, 1), (SEQ, 1)),
         jnp.kron(jnp.eye(SEQ, dtype=jnp.float32), W1[0].reshape(64, 1))],
        axis=1)
    r4 = (W1[1:5] + b1[None, :]).reshape(SEQ * 64, 1)
    w2b = jnp.kron(jnp.eye(SEQ, dtype=jnp.float32), W2[:, 0].reshape(1, 64))

    src_rows, emb_dst = _gather_call(packed, emb, src2d, dst2d)
    p, q = _mlp_call(src_rows, emb_dst, edge_attr, w4t, r4, w2b)

    dst_g = edge_index[1].reshape(n_edges // SW, SW)
    p_g = p.reshape(n_edges // SW, SW)
    q_g = q.reshape(n_edges // SW, SW)
    parts = _scatter_call(dst_g, p_g, q_g, n_nodes)

    return _combine_call(parts.reshape(32, 2, n_nodes), n_nodes)
